# Initial kernel scaffold; baseline (speedup 1.0000x reference)
import jax
import jax.numpy as jnp
from jax.experimental import pallas as pl


def kernel(X, E_X, snorm_n, snorm_e, edge_index, Wa, ba, Wb, bb, Wc, bc, Wd, bd, We, be, gamma_h, beta_h, gamma_e, beta_e):
    raise NotImplementedError("write your pallas kernel here")


# speedup vs baseline: 1.7029x; 1.7029x over previous
import functools

import jax
import jax.numpy as jnp
from jax import lax
from jax.experimental import pallas as pl
from jax.experimental.pallas import tpu as pltpu
from jax.experimental.pallas import tpu_sc as plsc

N = 10000
E = 320000
D = 128
HD = D // 2

NSUB = 16
ET = E // NSUB
K = 80
NCHUNK = ET // K
NPAD = 10240
NR = NPAD // NSUB
ZR = 128


def _proj_body(x_ref, wa_ref, ba_ref, wb_ref, bb_ref, wd_ref, bd_ref,
               we_ref, be_ref, ax_ref, sdb_ref, exp_ref):
    x = x_ref[...]
    ax_ref[...] = jnp.dot(x, wa_ref[...], preferred_element_type=jnp.float32) + ba_ref[...]
    bx = jnp.dot(x, wb_ref[...], preferred_element_type=jnp.float32) + bb_ref[...]
    dx = jnp.dot(x, wd_ref[...], preferred_element_type=jnp.float32) + bd_ref[...]
    sdb_ref[0] = jnp.concatenate([dx[:, :HD], bx[:, :HD]], axis=1)
    sdb_ref[1] = jnp.concatenate([dx[:, HD:], bx[:, HD:]], axis=1)
    ex = jnp.dot(x, we_ref[...], preferred_element_type=jnp.float32) + be_ref[...]
    exp_ref[0] = jnp.concatenate([ex[:, :HD], ex[:, :HD]], axis=1)
    exp_ref[1] = jnp.concatenate([ex[:, HD:], ex[:, HD:]], axis=1)


def _proj(X, Wa, ba, Wb, bb, Wd, bd, We, be):
    nb = 2000
    grid = (N // nb,)
    wspec = pl.BlockSpec((D, D), lambda i: (0, 0))
    bspec = pl.BlockSpec((1, D), lambda i: (0, 0))
    hspec = pl.BlockSpec((2, nb, D), lambda i: (0, i, 0))
    return pl.pallas_call(
        _proj_body,
        grid=grid,
        in_specs=[pl.BlockSpec((nb, D), lambda i: (i, 0)),
                  wspec, bspec, wspec, bspec, wspec, bspec, wspec, bspec],
        out_specs=[pl.BlockSpec((nb, D), lambda i: (i, 0)), hspec, hspec],
        out_shape=[jax.ShapeDtypeStruct((N, D), jnp.float32),
                   jax.ShapeDtypeStruct((2, N, D), jnp.float32),
                   jax.ShapeDtypeStruct((2, N, D), jnp.float32)],
    )(X, Wa, ba.reshape(1, D), Wb, bb.reshape(1, D), Wd, bd.reshape(1, D),
      We, be.reshape(1, D))


def _ce_body(ex_ref, wc_ref, bc_ref, ces_ref):
    ce = jnp.dot(ex_ref[...], wc_ref[...], preferred_element_type=jnp.float32) + bc_ref[...]
    ces_ref[0] = ce[:, :HD]
    ces_ref[1] = ce[:, HD:]


def _ce(E_X, Wc, bc):
    eb = 4000
    return pl.pallas_call(
        _ce_body,
        grid=(E // eb,),
        in_specs=[pl.BlockSpec((eb, D), lambda i: (i, 0)),
                  pl.BlockSpec((D, D), lambda i: (0, 0)),
                  pl.BlockSpec((1, D), lambda i: (0, 0))],
        out_specs=[pl.BlockSpec((2, eb, HD), lambda i: (0, i, 0))],
        out_shape=[jax.ShapeDtypeStruct((2, E, HD), jnp.float32)],
    )(E_X, Wc, bc.reshape(1, D))[0]


def _edge_body(src_hbm, dst_hbm, snorm_hbm, ce_hbm, sdb_hbm, exp_hbm,
               u_hbm, cs_hbm, stats_hbm,
               src_v, dst_v, dstc_v, srcc_v, snorm_v,
               ce_v, sdb_v, exg_v, u_v, cs_v, stats_v, sem):
    c = lax.axis_index("c")
    s = lax.axis_index("s")
    tbase = s * ET
    coff_n = c * N
    eoff = c * E
    z16 = jnp.zeros((16,), jnp.float32)

    def _chunk(i, carry):
        base = tbase + i * K
        pltpu.sync_copy(src_hbm.at[pl.ds(base, K)], src_v)
        pltpu.sync_copy(dst_hbm.at[pl.ds(base, K)], dst_v)
        pltpu.sync_copy(snorm_hbm.at[pl.ds(base, K)], snorm_v.at[pl.ds(0, K)])
        pltpu.sync_copy(ce_hbm.at[pl.ds(eoff + base, K), :], ce_v)

        def _adj(j, _):
            sl = pl.ds(j * 16, 16)
            srcc_v[sl] = src_v[sl] + coff_n
            dstc_v[sl] = dst_v[sl] + coff_n
            return 0

        lax.fori_loop(0, K // 16, _adj, 0)

        cp1 = pltpu.async_copy(sdb_hbm.at[srcc_v], sdb_v, sem)
        cp2 = pltpu.async_copy(exp_hbm.at[dstc_v], exg_v, sem)
        cp1.wait()
        cp2.wait()

        def _row(k, st):
            sn = snorm_v[pl.ds(k, 16)][0]
            st = list(st)
            for j in range(HD // 16):
                sl = pl.ds(j * 16, 16)
                e = ce_v[k, sl] + sdb_v[k, sl] + exg_v[k, sl]
                sg = 1.0 / (1.0 + jnp.exp(-e))
                u = e * sn
                u_v[k, sl] = u
                cs_v[k, sl] = sg * sdb_v[k, pl.ds(HD + j * 16, 16)]
                cs_v[k, pl.ds(HD + j * 16, 16)] = sg
                st[j] = st[j] + u
                st[4 + j] = st[4 + j] + u * u
            return tuple(st)

        carry = lax.fori_loop(0, K, _row, carry)

        pltpu.sync_copy(u_v, u_hbm.at[pl.ds(eoff + base, K), :])
        pltpu.sync_copy(cs_v, cs_hbm.at[pl.ds(eoff + base, K), :])
        return carry

    zeros8 = tuple(jnp.zeros((16,), jnp.float32) for _ in range(8))
    stats = lax.fori_loop(0, NCHUNK, _chunk, zeros8)
    for r in range(8):
        for j in range(8):
            stats_v[r, pl.ds(j * 16, 16)] = z16
    for j in range(4):
        stats_v[0, pl.ds(j * 16, 16)] = stats[j]
        stats_v[0, pl.ds(64 + j * 16, 16)] = stats[4 + j]
    pltpu.sync_copy(stats_v, stats_hbm.at[c, s])


def _edge(src, dst, snorm_e, ces, sdb, exp_):
    mesh = plsc.VectorSubcoreMesh(core_axis_name="c", subcore_axis_name="s",
                                  num_cores=2, num_subcores=NSUB)
    f = pl.kernel(
        _edge_body,
        mesh=mesh,
        out_type=[jax.ShapeDtypeStruct((2 * E, HD), jnp.float32),
                  jax.ShapeDtypeStruct((2 * E, D), jnp.float32),
                  jax.ShapeDtypeStruct((2, NSUB, 8, 2 * HD), jnp.float32)],
        scratch_types=[
            pltpu.VMEM((K,), jnp.int32),
            pltpu.VMEM((K,), jnp.int32),
            pltpu.VMEM((K,), jnp.int32),
            pltpu.VMEM((K,), jnp.int32),
            pltpu.VMEM((K + 16,), jnp.float32),
            pltpu.VMEM((K, HD), jnp.float32),
            pltpu.VMEM((K, D), jnp.float32),
            pltpu.VMEM((K, D), jnp.float32),
            pltpu.VMEM((K, HD), jnp.float32),
            pltpu.VMEM((K, D), jnp.float32),
            pltpu.VMEM((8, 2 * HD), jnp.float32),
            pltpu.SemaphoreType.DMA,
        ],
    )
    return f(src, dst, snorm_e,
             ces.reshape(2 * E, HD), sdb.reshape(2 * N, D),
             exp_.reshape(2 * N, D))


def _scat_body(dst_hbm, cs_hbm, nd_hbm, dst_v, cs_v, zero_v, acc_sh):
    s = lax.axis_index("s")
    tbase = s * ET
    nrow0 = s * NR
    z16 = jnp.zeros((16,), jnp.float32)

    def _zrow(r, _):
        for j in range(D // 16):
            zero_v[r, pl.ds(j * 16, 16)] = z16
        return 0

    lax.fori_loop(0, ZR, _zrow, 0)

    for p in range(2):
        for q in range(NR // ZR):
            pltpu.sync_copy(zero_v, acc_sh.at[pl.ds(nrow0 + q * ZR, ZR), :])
        plsc.subcore_barrier()

        eoff = p * E

        def _chunk(i, _):
            base = tbase + i * K
            pltpu.sync_copy(dst_hbm.at[pl.ds(base, K)], dst_v)
            pltpu.sync_copy(cs_hbm.at[pl.ds(eoff + base, K), :], cs_v)
            pltpu.sync_copy(cs_v, acc_sh.at[dst_v], add=True)
            return 0

        lax.fori_loop(0, NCHUNK, _chunk, 0)
        plsc.subcore_barrier()
        pltpu.sync_copy(acc_sh.at[pl.ds(nrow0, NR), :],
                        nd_hbm.at[pl.ds(p * NPAD + nrow0, NR), :])


def _scat(dst, cs):
    mesh = plsc.VectorSubcoreMesh(core_axis_name="c", subcore_axis_name="s",
                                  num_cores=1, num_subcores=NSUB)
    f = pl.kernel(
        _scat_body,
        mesh=mesh,
        out_type=[jax.ShapeDtypeStruct((2 * NPAD, D), jnp.float32)],
        scratch_types=[
            pltpu.VMEM((K,), jnp.int32),
            pltpu.VMEM((K, D), jnp.float32),
            pltpu.VMEM((ZR, D), jnp.float32),
            pltpu.VMEM_SHARED((NPAD, D), jnp.float32),
        ],
    )
    return f(dst, cs)[0]


def _enew_body(ex_ref, u_ref, stats_ref, g_ref, b_ref, out_ref):
    st = stats_ref[...].reshape(2, NSUB * 8, 2 * HD)
    tot = jnp.sum(st, axis=1)
    mean = jnp.concatenate([tot[0, :HD], tot[1, :HD]]) * (1.0 / E)
    msq = jnp.concatenate([tot[0, HD:], tot[1, HD:]]) * (1.0 / E)
    var = msq - mean * mean
    u = jnp.concatenate([u_ref[0], u_ref[1]], axis=1)
    bn = g_ref[...] * (u - mean[None, :]) / jnp.sqrt(var[None, :] + 1e-5) + b_ref[...]
    out_ref[...] = ex_ref[...] + jnp.maximum(bn, 0.0)


def _enew(E_X, u, stats, gamma_e, beta_e):
    eb = 4000
    return pl.pallas_call(
        _enew_body,
        grid=(E // eb,),
        in_specs=[pl.BlockSpec((eb, D), lambda i: (i, 0)),
                  pl.BlockSpec((2, eb, HD), lambda i: (0, i, 0)),
                  pl.BlockSpec((2 * NSUB * 8, 2 * HD), lambda i: (0, 0)),
                  pl.BlockSpec((1, D), lambda i: (0, 0)),
                  pl.BlockSpec((1, D), lambda i: (0, 0))],
        out_specs=[pl.BlockSpec((eb, D), lambda i: (i, 0))],
        out_shape=[jax.ShapeDtypeStruct((E, D), jnp.float32)],
    )(E_X, u.reshape(2, E, HD), stats.reshape(2 * NSUB * 8, 2 * HD),
      gamma_e.reshape(1, D), beta_e.reshape(1, D))[0]


def _h_body(x_ref, ax_ref, num_ref, den_ref, sn_ref, g_ref, b_ref, out_ref):
    num = jnp.concatenate([num_ref[0], num_ref[1]], axis=1)
    den = jnp.concatenate([den_ref[0], den_ref[1]], axis=1)
    has = den > 0.0
    hm = ax_ref[...] + num / jnp.where(has, den, 1.0)
    h0 = jnp.where(has, hm, x_ref[...]) * sn_ref[...]
    m = jnp.mean(h0, axis=0, keepdims=True)
    v = jnp.mean((h0 - m) * (h0 - m), axis=0, keepdims=True)
    h = g_ref[...] * (h0 - m) / jnp.sqrt(v + 1e-5) + b_ref[...]
    out_ref[...] = x_ref[...] + jnp.maximum(h, 0.0)


def _h(X, AX, num, den, snorm_n, gamma_h, beta_h):
    full2 = pl.BlockSpec((2, N, HD), lambda: (0, 0, 0))
    fullx = pl.BlockSpec((N, D), lambda: (0, 0))
    return pl.pallas_call(
        _h_body,
        in_specs=[fullx, fullx, full2, full2,
                  pl.BlockSpec((N, 1), lambda: (0, 0)),
                  pl.BlockSpec((1, D), lambda: (0, 0)),
                  pl.BlockSpec((1, D), lambda: (0, 0))],
        out_specs=[fullx],
        out_shape=[jax.ShapeDtypeStruct((N, D), jnp.float32)],
    )(X, AX, num, den, snorm_n,
      gamma_h.reshape(1, D), beta_h.reshape(1, D))[0]


def kernel(X, E_X, snorm_n, snorm_e, edge_index,
           Wa, ba, Wb, bb, Wc, bc, Wd, bd, We, be,
           gamma_h, beta_h, gamma_e, beta_e):
    src = edge_index[0].astype(jnp.int32)
    dst = edge_index[1].astype(jnp.int32)
    AX, sdb, exp_ = _proj(X, Wa, ba, Wb, bb, Wd, bd, We, be)
    ces = _ce(E_X, Wc, bc)
    u, cs, stats = _edge(src, dst, snorm_e.reshape(E), ces, sdb, exp_)
    numden = _scat(dst, cs).reshape(2, NPAD, D)
    E_new = _enew(E_X, u, stats, gamma_e, beta_e)
    num2 = numden[:, :N, :HD]
    den2 = numden[:, :N, HD:]
    H = _h(X, AX, num2, den2, snorm_n, gamma_h, beta_h)
    return (H, E_new)


# speedup vs baseline: 1.8794x; 1.1037x over previous
import functools

import jax
import jax.numpy as jnp
from jax import lax
from jax.experimental import pallas as pl
from jax.experimental.pallas import tpu as pltpu
from jax.experimental.pallas import tpu_sc as plsc

N = 10000
E = 320000
D = 128
HD = D // 2

NSUB = 16
ET = E // NSUB
K = 80
NCHUNK = ET // K
NPAD = 10240
NR = NPAD // NSUB
ZR = 128


def _proj_body(x_ref, wa_ref, ba_ref, wb_ref, bb_ref, wd_ref, bd_ref,
               we_ref, be_ref, ax_ref, sdb_ref, exp_ref):
    x = x_ref[...]
    ax_ref[...] = jnp.dot(x, wa_ref[...], preferred_element_type=jnp.float32) + ba_ref[...]
    bx = jnp.dot(x, wb_ref[...], preferred_element_type=jnp.float32) + bb_ref[...]
    dx = jnp.dot(x, wd_ref[...], preferred_element_type=jnp.float32) + bd_ref[...]
    sdb_ref[0] = jnp.concatenate([dx[:, :HD], bx[:, :HD]], axis=1)
    sdb_ref[1] = jnp.concatenate([dx[:, HD:], bx[:, HD:]], axis=1)
    ex = jnp.dot(x, we_ref[...], preferred_element_type=jnp.float32) + be_ref[...]
    exp_ref[0] = jnp.concatenate([ex[:, :HD], ex[:, :HD]], axis=1)
    exp_ref[1] = jnp.concatenate([ex[:, HD:], ex[:, HD:]], axis=1)


def _proj(X, Wa, ba, Wb, bb, Wd, bd, We, be):
    nb = 2000
    grid = (N // nb,)
    wspec = pl.BlockSpec((D, D), lambda i: (0, 0))
    bspec = pl.BlockSpec((1, D), lambda i: (0, 0))
    hspec = pl.BlockSpec((2, nb, D), lambda i: (0, i, 0))
    return pl.pallas_call(
        _proj_body,
        grid=grid,
        in_specs=[pl.BlockSpec((nb, D), lambda i: (i, 0)),
                  wspec, bspec, wspec, bspec, wspec, bspec, wspec, bspec],
        out_specs=[pl.BlockSpec((nb, D), lambda i: (i, 0)), hspec, hspec],
        out_shape=[jax.ShapeDtypeStruct((N, D), jnp.float32),
                   jax.ShapeDtypeStruct((2, N, D), jnp.float32),
                   jax.ShapeDtypeStruct((2, N, D), jnp.float32)],
    )(X, Wa, ba.reshape(1, D), Wb, bb.reshape(1, D), Wd, bd.reshape(1, D),
      We, be.reshape(1, D))


def _ce_body(ex_ref, wc_ref, bc_ref, ces_ref):
    ce = jnp.dot(ex_ref[...], wc_ref[...], preferred_element_type=jnp.float32) + bc_ref[...]
    ces_ref[0] = ce[:, :HD]
    ces_ref[1] = ce[:, HD:]


def _ce(E_X, Wc, bc):
    eb = 4000
    return pl.pallas_call(
        _ce_body,
        grid=(E // eb,),
        in_specs=[pl.BlockSpec((eb, D), lambda i: (i, 0)),
                  pl.BlockSpec((D, D), lambda i: (0, 0)),
                  pl.BlockSpec((1, D), lambda i: (0, 0))],
        out_specs=[pl.BlockSpec((2, eb, HD), lambda i: (0, i, 0))],
        out_shape=[jax.ShapeDtypeStruct((2, E, HD), jnp.float32)],
    )(E_X, Wc, bc.reshape(1, D))[0]


K2 = 128
CHT = E // K2 // NSUB
CHREM = E // K2 - CHT * NSUB


def _edge_body(srcc_hbm, dstc_hbm, ce_hbm, sdb_hbm, exp_hbm,
               e_hbm, cs_hbm,
               srcc_v, dstc_v, ce_v, sdb_v, exg_v, e_v, cs_v, sem):
    c = lax.axis_index("c")
    s = lax.axis_index("s")
    g0 = s * CHT + jnp.minimum(s, CHREM)
    nch = jnp.where(s < CHREM, CHT + 1, CHT)
    eoff = c * E

    def _chunk(i, _):
        base = (g0 + i) * K2
        pltpu.sync_copy(srcc_hbm.at[pl.ds(eoff + base, K2)], srcc_v)
        pltpu.sync_copy(dstc_hbm.at[pl.ds(eoff + base, K2)], dstc_v)
        pltpu.sync_copy(ce_hbm.at[pl.ds(eoff + base, K2), :], ce_v)

        cp1 = pltpu.async_copy(sdb_hbm.at[srcc_v], sdb_v, sem)
        cp2 = pltpu.async_copy(exp_hbm.at[dstc_v], exg_v, sem)
        cp1.wait()
        cp2.wait()

        def _row(k, _):
            for j in range(HD // 16):
                sl = pl.ds(j * 16, 16)
                e = ce_v[k, sl] + sdb_v[k, sl] + exg_v[k, sl]
                sg = 1.0 / (1.0 + jnp.exp(-e))
                e_v[k, sl] = e
                cs_v[k, sl] = sg * sdb_v[k, pl.ds(HD + j * 16, 16)]
                cs_v[k, pl.ds(HD + j * 16, 16)] = sg
            return 0

        lax.fori_loop(0, K2, _row, 0)

        pltpu.sync_copy(e_v, e_hbm.at[pl.ds(eoff + base, K2), :])
        pltpu.sync_copy(cs_v, cs_hbm.at[pl.ds(eoff + base, K2), :])
        return 0

    lax.fori_loop(0, nch, _chunk, 0)


def _edge(srcc, dstc, ces, sdb, exp_):
    mesh = plsc.VectorSubcoreMesh(core_axis_name="c", subcore_axis_name="s",
                                  num_cores=2, num_subcores=NSUB)
    f = pl.kernel(
        _edge_body,
        mesh=mesh,
        out_type=[jax.ShapeDtypeStruct((2 * E, HD), jnp.float32),
                  jax.ShapeDtypeStruct((2 * E, D), jnp.float32)],
        scratch_types=[
            pltpu.VMEM((K2,), jnp.int32),
            pltpu.VMEM((K2,), jnp.int32),
            pltpu.VMEM((K2, HD), jnp.float32),
            pltpu.VMEM((K2, D), jnp.float32),
            pltpu.VMEM((K2, D), jnp.float32),
            pltpu.VMEM((K2, HD), jnp.float32),
            pltpu.VMEM((K2, D), jnp.float32),
            pltpu.SemaphoreType.DMA,
        ],
    )
    return f(srcc, dstc, ces.reshape(2 * E, HD), sdb.reshape(2 * N, D),
             exp_.reshape(2 * N, D))


def _scat_body(dst_hbm, cs_hbm, nd_hbm, dst_v, cs_v, zero_v, acc_sh):
    s = lax.axis_index("s")
    tbase = s * ET
    nrow0 = s * NR
    z16 = jnp.zeros((16,), jnp.float32)

    def _zrow(r, _):
        for j in range(D // 16):
            zero_v[r, pl.ds(j * 16, 16)] = z16
        return 0

    lax.fori_loop(0, ZR, _zrow, 0)

    for p in range(2):
        for q in range(NR // ZR):
            pltpu.sync_copy(zero_v, acc_sh.at[pl.ds(nrow0 + q * ZR, ZR), :])
        plsc.subcore_barrier()

        eoff = p * E

        def _chunk(i, _):
            base = tbase + i * K
            pltpu.sync_copy(dst_hbm.at[pl.ds(base, K)], dst_v)
            pltpu.sync_copy(cs_hbm.at[pl.ds(eoff + base, K), :], cs_v)
            pltpu.sync_copy(cs_v, acc_sh.at[dst_v], add=True)
            return 0

        lax.fori_loop(0, NCHUNK, _chunk, 0)
        plsc.subcore_barrier()
        pltpu.sync_copy(acc_sh.at[pl.ds(nrow0, NR), :],
                        nd_hbm.at[pl.ds(p * NPAD + nrow0, NR), :])


def _scat(dst, cs):
    mesh = plsc.VectorSubcoreMesh(core_axis_name="c", subcore_axis_name="s",
                                  num_cores=1, num_subcores=NSUB)
    f = pl.kernel(
        _scat_body,
        mesh=mesh,
        out_type=[jax.ShapeDtypeStruct((2 * NPAD, D), jnp.float32)],
        scratch_types=[
            pltpu.VMEM((K,), jnp.int32),
            pltpu.VMEM((K, D), jnp.float32),
            pltpu.VMEM((ZR, D), jnp.float32),
            pltpu.VMEM_SHARED((NPAD, D), jnp.float32),
        ],
    )
    return f(dst, cs)[0]


def _ebn_body(e_ref, sn_ref, stats_ref):
    i = pl.program_id(0)

    @pl.when(i == 0)
    def _():
        stats_ref[...] = jnp.zeros_like(stats_ref)

    sn = sn_ref[...]
    u0 = e_ref[0] * sn
    u1 = e_ref[1] * sn
    s0 = jnp.concatenate([jnp.sum(u0, axis=0), jnp.sum(u0 * u0, axis=0)])
    s1 = jnp.concatenate([jnp.sum(u1, axis=0), jnp.sum(u1 * u1, axis=0)])
    stats_ref[...] += jnp.stack([s0, s1])


def _ebn(e, snorm_e):
    eb = 4000
    return pl.pallas_call(
        _ebn_body,
        grid=(E // eb,),
        in_specs=[pl.BlockSpec((2, eb, HD), lambda i: (0, i, 0)),
                  pl.BlockSpec((eb, 1), lambda i: (i, 0))],
        out_specs=[pl.BlockSpec((2, D), lambda i: (0, 0))],
        out_shape=[jax.ShapeDtypeStruct((2, D), jnp.float32)],
    )(e.reshape(2, E, HD), snorm_e)[0]


def _enew_body(ex_ref, e_ref, sn_ref, stats_ref, g_ref, b_ref, out_ref):
    st = stats_ref[...]
    mean = jnp.concatenate([st[0, :HD], st[1, :HD]]) * (1.0 / E)
    msq = jnp.concatenate([st[0, HD:], st[1, HD:]]) * (1.0 / E)
    var = msq - mean * mean
    u = jnp.concatenate([e_ref[0], e_ref[1]], axis=1) * sn_ref[...]
    bn = g_ref[...] * (u - mean[None, :]) / jnp.sqrt(var[None, :] + 1e-5) + b_ref[...]
    out_ref[...] = ex_ref[...] + jnp.maximum(bn, 0.0)


def _enew(E_X, e, snorm_e, stats, gamma_e, beta_e):
    eb = 4000
    return pl.pallas_call(
        _enew_body,
        grid=(E // eb,),
        in_specs=[pl.BlockSpec((eb, D), lambda i: (i, 0)),
                  pl.BlockSpec((2, eb, HD), lambda i: (0, i, 0)),
                  pl.BlockSpec((eb, 1), lambda i: (i, 0)),
                  pl.BlockSpec((2, D), lambda i: (0, 0)),
                  pl.BlockSpec((1, D), lambda i: (0, 0)),
                  pl.BlockSpec((1, D), lambda i: (0, 0))],
        out_specs=[pl.BlockSpec((eb, D), lambda i: (i, 0))],
        out_shape=[jax.ShapeDtypeStruct((E, D), jnp.float32)],
    )(E_X, e.reshape(2, E, HD), snorm_e, stats,
      gamma_e.reshape(1, D), beta_e.reshape(1, D))[0]


def _h_body(x_ref, ax_ref, num_ref, den_ref, sn_ref, g_ref, b_ref, out_ref):
    num = jnp.concatenate([num_ref[0], num_ref[1]], axis=1)
    den = jnp.concatenate([den_ref[0], den_ref[1]], axis=1)
    has = den > 0.0
    hm = ax_ref[...] + num / jnp.where(has, den, 1.0)
    h0 = jnp.where(has, hm, x_ref[...]) * sn_ref[...]
    m = jnp.mean(h0, axis=0, keepdims=True)
    v = jnp.mean((h0 - m) * (h0 - m), axis=0, keepdims=True)
    h = g_ref[...] * (h0 - m) / jnp.sqrt(v + 1e-5) + b_ref[...]
    out_ref[...] = x_ref[...] + jnp.maximum(h, 0.0)


def _h(X, AX, num, den, snorm_n, gamma_h, beta_h):
    full2 = pl.BlockSpec((2, N, HD), lambda: (0, 0, 0))
    fullx = pl.BlockSpec((N, D), lambda: (0, 0))
    return pl.pallas_call(
        _h_body,
        in_specs=[fullx, fullx, full2, full2,
                  pl.BlockSpec((N, 1), lambda: (0, 0)),
                  pl.BlockSpec((1, D), lambda: (0, 0)),
                  pl.BlockSpec((1, D), lambda: (0, 0))],
        out_specs=[fullx],
        out_shape=[jax.ShapeDtypeStruct((N, D), jnp.float32)],
    )(X, AX, num, den, snorm_n,
      gamma_h.reshape(1, D), beta_h.reshape(1, D))[0]


def kernel(X, E_X, snorm_n, snorm_e, edge_index,
           Wa, ba, Wb, bb, Wc, bc, Wd, bd, We, be,
           gamma_h, beta_h, gamma_e, beta_e):
    src = edge_index[0].astype(jnp.int32)
    dst = edge_index[1].astype(jnp.int32)
    srcc = jnp.concatenate([src, src + N])
    dstc = jnp.concatenate([dst, dst + N])
    AX, sdb, exp_ = _proj(X, Wa, ba, Wb, bb, Wd, bd, We, be)
    ces = _ce(E_X, Wc, bc)
    e, cs = _edge(srcc, dstc, ces, sdb, exp_)
    numden = _scat(dst, cs).reshape(2, NPAD, D)
    stats = _ebn(e, snorm_e)
    E_new = _enew(E_X, e, snorm_e, stats, gamma_e, beta_e)
    num2 = numden[:, :N, :HD]
    den2 = numden[:, :N, HD:]
    H = _h(X, AX, num2, den2, snorm_n, gamma_h, beta_h)
    return (H, E_new)


# speedup vs baseline: 2.0067x; 1.0677x over previous
import functools

import jax
import jax.numpy as jnp
from jax import lax
from jax.experimental import pallas as pl
from jax.experimental.pallas import tpu as pltpu
from jax.experimental.pallas import tpu_sc as plsc

N = 10000
E = 320000
D = 128
HD = D // 2

NSUB = 16
ET = E // NSUB
K = 80
NCHUNK = ET // K
NPAD = 10240
NR = NPAD // NSUB
ZR = 128


def _proj_body(x_ref, wa_ref, ba_ref, wb_ref, bb_ref, wd_ref, bd_ref,
               we_ref, be_ref, ax_ref, sdb_ref, exp_ref):
    x = x_ref[...]
    ax_ref[...] = jnp.dot(x, wa_ref[...], preferred_element_type=jnp.float32) + ba_ref[...]
    bx = jnp.dot(x, wb_ref[...], preferred_element_type=jnp.float32) + bb_ref[...]
    dx = jnp.dot(x, wd_ref[...], preferred_element_type=jnp.float32) + bd_ref[...]
    sdb_ref[0] = jnp.concatenate([dx[:, :HD], bx[:, :HD]], axis=1)
    sdb_ref[1] = jnp.concatenate([dx[:, HD:], bx[:, HD:]], axis=1)
    ex = jnp.dot(x, we_ref[...], preferred_element_type=jnp.float32) + be_ref[...]
    exp_ref[0] = jnp.concatenate([ex[:, :HD], ex[:, :HD]], axis=1)
    exp_ref[1] = jnp.concatenate([ex[:, HD:], ex[:, HD:]], axis=1)


def _proj(X, Wa, ba, Wb, bb, Wd, bd, We, be):
    nb = 2000
    grid = (N // nb,)
    wspec = pl.BlockSpec((D, D), lambda i: (0, 0))
    bspec = pl.BlockSpec((1, D), lambda i: (0, 0))
    hspec = pl.BlockSpec((2, nb, D), lambda i: (0, i, 0))
    return pl.pallas_call(
        _proj_body,
        grid=grid,
        in_specs=[pl.BlockSpec((nb, D), lambda i: (i, 0)),
                  wspec, bspec, wspec, bspec, wspec, bspec, wspec, bspec],
        out_specs=[pl.BlockSpec((nb, D), lambda i: (i, 0)), hspec, hspec],
        out_shape=[jax.ShapeDtypeStruct((N, D), jnp.float32),
                   jax.ShapeDtypeStruct((2, N, D), jnp.float32),
                   jax.ShapeDtypeStruct((2, N, D), jnp.float32)],
    )(X, Wa, ba.reshape(1, D), Wb, bb.reshape(1, D), Wd, bd.reshape(1, D),
      We, be.reshape(1, D))


def _ce_body(ex_ref, wc_ref, bc_ref, ces_ref):
    ce = jnp.dot(ex_ref[...], wc_ref[...], preferred_element_type=jnp.float32) + bc_ref[...]
    ces_ref[0] = ce[:, :HD]
    ces_ref[1] = ce[:, HD:]


def _ce(E_X, Wc, bc):
    eb = 4000
    return pl.pallas_call(
        _ce_body,
        grid=(E // eb,),
        in_specs=[pl.BlockSpec((eb, D), lambda i: (i, 0)),
                  pl.BlockSpec((D, D), lambda i: (0, 0)),
                  pl.BlockSpec((1, D), lambda i: (0, 0))],
        out_specs=[pl.BlockSpec((2, eb, HD), lambda i: (0, i, 0))],
        out_shape=[jax.ShapeDtypeStruct((2, E, HD), jnp.float32)],
    )(E_X, Wc, bc.reshape(1, D))[0]


K2 = 128
CHT = E // K2 // NSUB
CHREM = E // K2 - CHT * NSUB
NPAIR = CHT // 2


def _edge_body(srcc_hbm, dstc_hbm, ce_hbm, sdb_hbm, exp_hbm,
               e_hbm, cs_hbm,
               srcc_v, dstc_v, ce_v, sdb_v, exg_v, e_v, cs_v,
               isem, gsem):
    c = lax.axis_index("c")
    s = lax.axis_index("s")
    g0 = s * CHT
    eoff = c * E

    def ibase(il):
        return eoff + (g0 + jnp.minimum(il, CHT - 1)) * K2

    def iload(b, il):
        off = ibase(il)
        pltpu.sync_copy(srcc_hbm.at[pl.ds(off, K2)], srcc_v.at[b])
        pltpu.sync_copy(dstc_hbm.at[pl.ds(off, K2)], dstc_v.at[b])

    def gstart(b):
        pltpu.async_copy(sdb_hbm.at[srcc_v.at[b]], sdb_v.at[b], gsem)
        pltpu.async_copy(exp_hbm.at[dstc_v.at[b]], exg_v.at[b], gsem)

    def gwait(b):
        pltpu.make_async_copy(sdb_hbm.at[pl.ds(0, K2), :], sdb_v.at[b], gsem).wait()
        pltpu.make_async_copy(exp_hbm.at[pl.ds(0, K2), :], exg_v.at[b], gsem).wait()

    def compute(b, il):
        off = ibase(il)
        pltpu.sync_copy(ce_hbm.at[pl.ds(off, K2), :], ce_v)

        def _row(k, _):
            for j in range(HD // 16):
                sl = pl.ds(j * 16, 16)
                e = ce_v[k, sl] + sdb_v[b, k, sl] + exg_v[b, k, sl]
                sg = 1.0 / (1.0 + jnp.exp(-e))
                e_v[k, sl] = e
                cs_v[k, sl] = sg * sdb_v[b, k, pl.ds(HD + j * 16, 16)]
                cs_v[k, pl.ds(HD + j * 16, 16)] = sg
            return 0
        lax.fori_loop(0, K2, _row, 0)
        pltpu.sync_copy(e_v, e_hbm.at[pl.ds(off, K2), :])
        pltpu.sync_copy(cs_v, cs_hbm.at[pl.ds(off, K2), :])

    iload(0, 0)
    gstart(0)

    def _pair(i, _):
        c0 = 2 * i
        iload(1, c0 + 1)
        gstart(1)
        gwait(0)
        compute(0, c0)

        iload(0, c0 + 2)
        gstart(0)
        gwait(1)
        compute(1, c0 + 1)
        return 0

    lax.fori_loop(0, NPAIR, _pair, 0)
    gwait(0)

    @pl.when(s < CHREM)
    def _():
        off = eoff + (NSUB * CHT + s) * K2
        pltpu.sync_copy(srcc_hbm.at[pl.ds(off, K2)], srcc_v.at[0])
        pltpu.sync_copy(dstc_hbm.at[pl.ds(off, K2)], dstc_v.at[0])
        gstart(0)
        gwait(0)
        pltpu.sync_copy(ce_hbm.at[pl.ds(off, K2), :], ce_v)

        def _row(k, _):
            for j in range(HD // 16):
                sl = pl.ds(j * 16, 16)
                e = ce_v[k, sl] + sdb_v[0, k, sl] + exg_v[0, k, sl]
                sg = 1.0 / (1.0 + jnp.exp(-e))
                e_v[k, sl] = e
                cs_v[k, sl] = sg * sdb_v[0, k, pl.ds(HD + j * 16, 16)]
                cs_v[k, pl.ds(HD + j * 16, 16)] = sg
            return 0
        lax.fori_loop(0, K2, _row, 0)
        pltpu.sync_copy(e_v, e_hbm.at[pl.ds(off, K2), :])
        pltpu.sync_copy(cs_v, cs_hbm.at[pl.ds(off, K2), :])


def _edge(srcc, dstc, ces, sdb, exp_):
    mesh = plsc.VectorSubcoreMesh(core_axis_name="c", subcore_axis_name="s",
                                  num_cores=2, num_subcores=NSUB)
    f = pl.kernel(
        _edge_body,
        mesh=mesh,
        out_type=[jax.ShapeDtypeStruct((2 * E, HD), jnp.float32),
                  jax.ShapeDtypeStruct((2 * E, D), jnp.float32)],
        scratch_types=[
            pltpu.VMEM((2, K2), jnp.int32),
            pltpu.VMEM((2, K2), jnp.int32),
            pltpu.VMEM((K2, HD), jnp.float32),
            pltpu.VMEM((2, K2, D), jnp.float32),
            pltpu.VMEM((2, K2, D), jnp.float32),
            pltpu.VMEM((K2, HD), jnp.float32),
            pltpu.VMEM((K2, D), jnp.float32),
            pltpu.SemaphoreType.DMA,
            pltpu.SemaphoreType.DMA,
        ],
    )
    return f(srcc, dstc, ces.reshape(2 * E, HD), sdb.reshape(2 * N, D),
             exp_.reshape(2 * N, D))


def _scat_body(dst_hbm, cs_hbm, nd_hbm, dst_v, cs_v, zero_v, acc_sh):
    s = lax.axis_index("s")
    tbase = s * ET
    nrow0 = s * NR
    z16 = jnp.zeros((16,), jnp.float32)

    def _zrow(r, _):
        for j in range(D // 16):
            zero_v[r, pl.ds(j * 16, 16)] = z16
        return 0

    lax.fori_loop(0, ZR, _zrow, 0)

    for p in range(2):
        for q in range(NR // ZR):
            pltpu.sync_copy(zero_v, acc_sh.at[pl.ds(nrow0 + q * ZR, ZR), :])
        plsc.subcore_barrier()

        eoff = p * E

        def _chunk(i, _):
            base = tbase + i * K
            pltpu.sync_copy(dst_hbm.at[pl.ds(base, K)], dst_v)
            pltpu.sync_copy(cs_hbm.at[pl.ds(eoff + base, K), :], cs_v)
            pltpu.sync_copy(cs_v, acc_sh.at[dst_v], add=True)
            return 0

        lax.fori_loop(0, NCHUNK, _chunk, 0)
        plsc.subcore_barrier()
        pltpu.sync_copy(acc_sh.at[pl.ds(nrow0, NR), :],
                        nd_hbm.at[pl.ds(p * NPAD + nrow0, NR), :])


def _scat(dst, cs):
    mesh = plsc.VectorSubcoreMesh(core_axis_name="c", subcore_axis_name="s",
                                  num_cores=1, num_subcores=NSUB)
    f = pl.kernel(
        _scat_body,
        mesh=mesh,
        out_type=[jax.ShapeDtypeStruct((2 * NPAD, D), jnp.float32)],
        scratch_types=[
            pltpu.VMEM((K,), jnp.int32),
            pltpu.VMEM((K, D), jnp.float32),
            pltpu.VMEM((ZR, D), jnp.float32),
            pltpu.VMEM_SHARED((NPAD, D), jnp.float32),
        ],
    )
    return f(dst, cs)[0]


def _ebn_body(e_ref, sn_ref, stats_ref):
    i = pl.program_id(0)

    @pl.when(i == 0)
    def _():
        stats_ref[...] = jnp.zeros_like(stats_ref)

    sn = sn_ref[...]
    u0 = e_ref[0] * sn
    u1 = e_ref[1] * sn
    s0 = jnp.concatenate([jnp.sum(u0, axis=0), jnp.sum(u0 * u0, axis=0)])
    s1 = jnp.concatenate([jnp.sum(u1, axis=0), jnp.sum(u1 * u1, axis=0)])
    stats_ref[...] += jnp.stack([s0, s1])


def _ebn(e, snorm_e):
    eb = 4000
    return pl.pallas_call(
        _ebn_body,
        grid=(E // eb,),
        in_specs=[pl.BlockSpec((2, eb, HD), lambda i: (0, i, 0)),
                  pl.BlockSpec((eb, 1), lambda i: (i, 0))],
        out_specs=[pl.BlockSpec((2, D), lambda i: (0, 0))],
        out_shape=[jax.ShapeDtypeStruct((2, D), jnp.float32)],
    )(e.reshape(2, E, HD), snorm_e)[0]


def _enew_body(ex_ref, e_ref, sn_ref, stats_ref, g_ref, b_ref, out_ref):
    st = stats_ref[...]
    mean = jnp.concatenate([st[0, :HD], st[1, :HD]]) * (1.0 / E)
    msq = jnp.concatenate([st[0, HD:], st[1, HD:]]) * (1.0 / E)
    var = msq - mean * mean
    u = jnp.concatenate([e_ref[0], e_ref[1]], axis=1) * sn_ref[...]
    bn = g_ref[...] * (u - mean[None, :]) / jnp.sqrt(var[None, :] + 1e-5) + b_ref[...]
    out_ref[...] = ex_ref[...] + jnp.maximum(bn, 0.0)


def _enew(E_X, e, snorm_e, stats, gamma_e, beta_e):
    eb = 4000
    return pl.pallas_call(
        _enew_body,
        grid=(E // eb,),
        in_specs=[pl.BlockSpec((eb, D), lambda i: (i, 0)),
                  pl.BlockSpec((2, eb, HD), lambda i: (0, i, 0)),
                  pl.BlockSpec((eb, 1), lambda i: (i, 0)),
                  pl.BlockSpec((2, D), lambda i: (0, 0)),
                  pl.BlockSpec((1, D), lambda i: (0, 0)),
                  pl.BlockSpec((1, D), lambda i: (0, 0))],
        out_specs=[pl.BlockSpec((eb, D), lambda i: (i, 0))],
        out_shape=[jax.ShapeDtypeStruct((E, D), jnp.float32)],
    )(E_X, e.reshape(2, E, HD), snorm_e, stats,
      gamma_e.reshape(1, D), beta_e.reshape(1, D))[0]


def _h_body(x_ref, ax_ref, num_ref, den_ref, sn_ref, g_ref, b_ref, out_ref):
    num = jnp.concatenate([num_ref[0], num_ref[1]], axis=1)
    den = jnp.concatenate([den_ref[0], den_ref[1]], axis=1)
    has = den > 0.0
    hm = ax_ref[...] + num / jnp.where(has, den, 1.0)
    h0 = jnp.where(has, hm, x_ref[...]) * sn_ref[...]
    m = jnp.mean(h0, axis=0, keepdims=True)
    v = jnp.mean((h0 - m) * (h0 - m), axis=0, keepdims=True)
    h = g_ref[...] * (h0 - m) / jnp.sqrt(v + 1e-5) + b_ref[...]
    out_ref[...] = x_ref[...] + jnp.maximum(h, 0.0)


def _h(X, AX, num, den, snorm_n, gamma_h, beta_h):
    full2 = pl.BlockSpec((2, N, HD), lambda: (0, 0, 0))
    fullx = pl.BlockSpec((N, D), lambda: (0, 0))
    return pl.pallas_call(
        _h_body,
        in_specs=[fullx, fullx, full2, full2,
                  pl.BlockSpec((N, 1), lambda: (0, 0)),
                  pl.BlockSpec((1, D), lambda: (0, 0)),
                  pl.BlockSpec((1, D), lambda: (0, 0))],
        out_specs=[fullx],
        out_shape=[jax.ShapeDtypeStruct((N, D), jnp.float32)],
    )(X, AX, num, den, snorm_n,
      gamma_h.reshape(1, D), beta_h.reshape(1, D))[0]


def kernel(X, E_X, snorm_n, snorm_e, edge_index,
           Wa, ba, Wb, bb, Wc, bc, Wd, bd, We, be,
           gamma_h, beta_h, gamma_e, beta_e):
    src = edge_index[0].astype(jnp.int32)
    dst = edge_index[1].astype(jnp.int32)
    srcc = jnp.concatenate([src, src + N])
    dstc = jnp.concatenate([dst, dst + N])
    AX, sdb, exp_ = _proj(X, Wa, ba, Wb, bb, Wd, bd, We, be)
    ces = _ce(E_X, Wc, bc)
    e, cs = _edge(srcc, dstc, ces, sdb, exp_)
    numden = _scat(dst, cs).reshape(2, NPAD, D)
    stats = _ebn(e, snorm_e)
    E_new = _enew(E_X, e, snorm_e, stats, gamma_e, beta_e)
    num2 = numden[:, :N, :HD]
    den2 = numden[:, :N, HD:]
    H = _h(X, AX, num2, den2, snorm_n, gamma_h, beta_h)
    return (H, E_new)
